# baseline (speedup 1.0000x reference)
import jax
import jax.numpy as jnp
from jax import lax
from jax.experimental import pallas as pl

_BB = 4


def _body(x_ref, der_ref, out_ref):
    for i in range(_BB):
        xb = jnp.broadcast_to(x_ref[0, i], (8, x_ref.shape[-1]))
        y = lax.dot_general(
            xb, der_ref[i],
            dimension_numbers=(((1,), (1,)), ((), ())),
            preferred_element_type=jnp.float32,
        )
        y0 = y[0]
        out_ref[0, i] = y0 * y0


def kernel(x, der_desc_wrt_pos):
    B, A, D, T = der_desc_wrt_pos.shape
    der_t = der_desc_wrt_pos.transpose(0, 3, 1, 2).reshape(B, T * A, D)
    x3 = x.reshape(B // _BB, _BB, D)
    out = pl.pallas_call(
        _body,
        grid=(B // _BB,),
        in_specs=[
            pl.BlockSpec((1, _BB, D), lambda b: (b, 0, 0)),
            pl.BlockSpec((_BB, T * A, D), lambda b: (b, 0, 0)),
        ],
        out_specs=pl.BlockSpec((1, _BB, T * A), lambda b: (b, 0, 0)),
        out_shape=jax.ShapeDtypeStruct((B // _BB, _BB, T * A), jnp.float32),
    )(x3, der_t)
    return out.reshape(B, T, A).transpose(0, 2, 1).reshape(B, A * T)


# speedup vs baseline: 1.0704x; 1.0704x over previous
import jax
import jax.numpy as jnp
from jax import lax
from jax.experimental import pallas as pl

_BB = 16


def _body(x_ref, der_ref, out_ref):
    for i in range(_BB):
        xb = jnp.broadcast_to(x_ref[0, i], (8, x_ref.shape[-1]))
        y = lax.dot_general(
            xb, der_ref[i],
            dimension_numbers=(((1,), (1,)), ((), ())),
            preferred_element_type=jnp.float32,
        )
        y0 = y[0]
        out_ref[0, i] = y0 * y0


def kernel(x, der_desc_wrt_pos):
    B, A, D, T = der_desc_wrt_pos.shape
    der_t = der_desc_wrt_pos.transpose(0, 3, 1, 2).reshape(B, T * A, D)
    x3 = x.reshape(B // _BB, _BB, D)
    out = pl.pallas_call(
        _body,
        grid=(B // _BB,),
        in_specs=[
            pl.BlockSpec((1, _BB, D), lambda b: (b, 0, 0)),
            pl.BlockSpec((_BB, T * A, D), lambda b: (b, 0, 0)),
        ],
        out_specs=pl.BlockSpec((1, _BB, T * A), lambda b: (b, 0, 0)),
        out_shape=jax.ShapeDtypeStruct((B // _BB, _BB, T * A), jnp.float32),
    )(x3, der_t)
    return out.reshape(B, T, A).transpose(0, 2, 1).reshape(B, A * T)
